# baseline (speedup 1.0000x reference)
import functools

import jax
import jax.numpy as jnp
from jax import lax
from jax.experimental import pallas as pl
from jax.experimental.pallas import tpu as pltpu
from jax.experimental.pallas import tpu_sc as plsc

_B = 4
_C = 128
_LVL_DIMS = ((64, 64), (32, 32), (16, 16))
_LVL_SIZES = tuple(h * w for h, w in _LVL_DIMS)
_LEN = sum(_LVL_SIZES)
_STARTS = (0, _LVL_SIZES[0], _LVL_SIZES[0] + _LVL_SIZES[1])
_NH = 16
_NLVL = 3
_NPT = 4
_DH = _C // _NH
_NK = _NLVL * _NPT

_NQ = 128
_NCORES = 2
_NSUB = 16
_NW = _NCORES * _NSUB

_RPH = 48
_NROWS = _NH * _RPH


def _proj_kernel(x_ref, w_ref, b_ref, o_ref):
    y = jnp.dot(w_ref[...], x_ref[...], preferred_element_type=jnp.float32)
    o_ref[...] = y + b_ref[:, 0:1]


def _tc_project(x, w, bias2d, n_out_rows):
    nq_tile = 384
    return pl.pallas_call(
        _proj_kernel,
        grid=(_LEN // nq_tile,),
        in_specs=[
            pl.BlockSpec((_C, nq_tile), lambda j: (0, j)),
            pl.BlockSpec((n_out_rows, _C), lambda j: (0, 0)),
            pl.BlockSpec((n_out_rows, 128), lambda j: (0, 0)),
        ],
        out_specs=pl.BlockSpec((n_out_rows, nq_tile), lambda j: (0, j)),
        out_shape=jax.ShapeDtypeStruct((n_out_rows, _LEN), jnp.float32),
    )(x, w, bias2d)


def _out_kernel(x_ref, w_ref, b_ref, o_ref):
    y = jnp.dot(w_ref[...], x_ref[...], preferred_element_type=jnp.float32)
    o_ref[...] = y + b_ref[:, 0:1]


def _tc_out(x, w, bias2d):
    nq_tile = 384
    return pl.pallas_call(
        _out_kernel,
        grid=(_LEN // nq_tile,),
        in_specs=[
            pl.BlockSpec((_C, nq_tile), lambda j: (0, j)),
            pl.BlockSpec((_C, _C), lambda j: (0, 0)),
            pl.BlockSpec((_C, 128), lambda j: (0, 0)),
        ],
        out_specs=pl.BlockSpec((_C, nq_tile), lambda j: (0, j)),
        out_shape=jax.ShapeDtypeStruct((_C, _LEN), jnp.float32),
    )(x, w, bias2d)


def _sc_body(y_hbm, out_hbm, vtab, oa_v, out_v, in_sem, out_sem):
    wid = lax.axis_index("s") * _NCORES + lax.axis_index("c")
    h = wid // 2
    half = wid % 2

    def in_copy(h, q0, slot):
        return pltpu.make_async_copy(
            y_hbm.at[pl.ds(_RPH * h + _DH, 40), pl.ds(q0, _NQ)],
            oa_v.at[slot], in_sem)

    def out_copy(h, q0, slot):
        return pltpu.make_async_copy(
            out_v.at[slot], out_hbm.at[pl.ds(_DH * h, _DH), pl.ds(q0, _NQ)],
            out_sem)

    if True:
        pltpu.sync_copy(y_hbm.at[pl.ds(_RPH * h, _DH), :], vtab)

        for lq in range(_NLVL):
            hq, wq = _LVL_DIMS[lq]
            half_sz = _LVL_SIZES[lq] // 2
            n_chunks = half_sz // _NQ
            lg2w = wq.bit_length() - 1
            rel0 = half * half_sz

            in_copy(h, _STARTS[lq] + rel0, 0).start()

            def chunk_body(ci, carry2, lq=lq, hq=hq, wq=wq, lg2w=lg2w,
                           h=h, rel0=rel0, n_chunks=n_chunks):
                q0 = _STARTS[lq] + rel0 + ci * _NQ
                slot = lax.rem(ci, 2)

                @pl.when(ci + 1 < n_chunks)
                def _():
                    in_copy(h, q0 + _NQ, 1 - slot).start()

                in_copy(h, q0, slot).wait()

                @pl.when(ci >= 2)
                def _():
                    out_copy(h, q0 - 2 * _NQ, slot).wait()

                def do_group(s0):
                    rel = rel0 + ci * _NQ + s0 + lax.iota(jnp.int32, 16)
                    xq = rel & (wq - 1)
                    yq = lax.shift_right_logical(rel, lg2w)
                    xf = xq.astype(jnp.float32)
                    yf = yq.astype(jnp.float32)

                    avs = [oa_v[slot, 24 + k, pl.ds(s0, 16)] for k in range(_NK)]
                    m = avs[0]
                    for k in range(1, _NK):
                        m = jnp.maximum(m, avs[k])
                    es = [jnp.exp(a - m) for a in avs]
                    ssum = es[0]
                    for k in range(1, _NK):
                        ssum = ssum + es[k]
                    rs = 1.0 / ssum

                    acc = [jnp.zeros((16,), jnp.float32) for _ in range(_DH)]
                    for k in range(_NK):
                        l = k // _NPT
                        hl, wl = _LVL_DIMS[l]
                        base = _STARTS[l]
                        sx = wl / wq
                        sy = hl / hq
                        gx = xf * sx + (0.5 * sx - 0.5) + oa_v[slot, k, pl.ds(s0, 16)]
                        gy = yf * sy + (0.5 * sy - 0.5) + oa_v[slot, 12 + k, pl.ds(s0, 16)]

                        x0t = gx.astype(jnp.int32)
                        x0i = x0t - (x0t.astype(jnp.float32) > gx).astype(jnp.int32)
                        y0t = gy.astype(jnp.int32)
                        y0i = y0t - (y0t.astype(jnp.float32) > gy).astype(jnp.int32)
                        fx = gx - x0i.astype(jnp.float32)
                        fy = gy - y0i.astype(jnp.float32)

                        a = es[k] * rs
                        zero = jnp.zeros((16,), jnp.float32)
                        wx0 = jnp.where((x0i >= 0) & (x0i <= wl - 1), 1.0 - fx, zero)
                        wx1 = jnp.where((x0i >= -1) & (x0i < wl - 1), fx, zero)
                        wy0 = jnp.where((y0i >= 0) & (y0i <= hl - 1), (1.0 - fy) * a, zero)
                        wy1 = jnp.where((y0i >= -1) & (y0i < hl - 1), fy * a, zero)

                        xc0 = jnp.clip(x0i, 0, wl - 1)
                        xc1 = jnp.clip(x0i + 1, 0, wl - 1)
                        yc0 = jnp.clip(y0i, 0, hl - 1)
                        yc1 = jnp.clip(y0i + 1, 0, hl - 1)
                        row0 = base + yc0 * wl
                        row1 = base + yc1 * wl
                        i00 = row0 + xc0
                        i01 = row0 + xc1
                        i10 = row1 + xc0
                        i11 = row1 + xc1
                        w00 = wx0 * wy0
                        w01 = wx1 * wy0
                        w10 = wx0 * wy1
                        w11 = wx1 * wy1

                        for ch in range(_DH):
                            chv = jnp.full((16,), ch, jnp.int32)
                            g00 = plsc.load_gather(vtab, [chv, i00])
                            g01 = plsc.load_gather(vtab, [chv, i01])
                            g10 = plsc.load_gather(vtab, [chv, i10])
                            g11 = plsc.load_gather(vtab, [chv, i11])
                            acc[ch] = acc[ch] + (g00 * w00 + g01 * w01 +
                                                 g10 * w10 + g11 * w11)

                    for ch in range(_DH):
                        out_v[slot, ch, pl.ds(s0, 16)] = acc[ch]

                def sub_body(si, carry3):
                    do_group(si * 32)
                    do_group(si * 32 + 16)
                    return carry3

                lax.fori_loop(0, _NQ // 32, sub_body, 0)
                out_copy(h, q0, slot).start()
                return carry2

            lax.fori_loop(0, n_chunks, chunk_body, 0)
            for d in range(min(2, n_chunks)):
                out_copy(h, _STARTS[lq], 0).wait()


def _sc_sample(y):
    mesh = plsc.VectorSubcoreMesh(
        core_axis_name="c", subcore_axis_name="s",
        num_cores=_NCORES, num_subcores=_NSUB)
    k = pl.kernel(
        _sc_body,
        out_type=jax.ShapeDtypeStruct((_C, _LEN), jnp.float32),
        mesh=mesh,
        compiler_params=pltpu.CompilerParams(needs_layout_passes=False),
        scratch_types=[
            pltpu.VMEM((_DH, _LEN), jnp.float32),
            pltpu.VMEM((2, 40, _NQ), jnp.float32),
            pltpu.VMEM((2, _DH, _NQ), jnp.float32),
            pltpu.SemaphoreType.DMA,
            pltpu.SemaphoreType.DMA,
        ],
    )
    return k(y)


def kernel(feature_map1, feature_map2, feature_map3, W_value, b_value,
           W_off, b_off, W_attn, b_attn, W_out, b_out):
    x = jnp.concatenate(
        [feature_map1.reshape(_B, _C, -1),
         feature_map2.reshape(_B, _C, -1),
         feature_map3.reshape(_B, _C, -1)], axis=2)

    w_off_r = W_off.reshape(_NH, _NK, 2, _C)
    b_off_r = b_off.reshape(_NH, _NK, 2)
    w_all = jnp.concatenate(
        [W_value.reshape(_NH, _DH, _C),
         w_off_r[:, :, 0, :], w_off_r[:, :, 1, :],
         W_attn.reshape(_NH, _NK, _C),
         jnp.zeros((_NH, 4, _C), jnp.float32)], axis=1).reshape(_NROWS, _C)
    b_all = jnp.concatenate(
        [b_value.reshape(_NH, _DH),
         b_off_r[:, :, 0], b_off_r[:, :, 1],
         b_attn.reshape(_NH, _NK),
         jnp.zeros((_NH, 4), jnp.float32)], axis=1).reshape(_NROWS)
    bias2d = jnp.tile(b_all[:, None], (1, 128))

    bias_out = jnp.tile(b_out[:, None], (1, 128))
    os = []
    for b in range(_B):
        y_b = _tc_project(x[b], w_all, bias2d, _NROWS)
        s_b = _sc_sample(y_b)
        os.append(_tc_out(s_b, W_out, bias_out))
    o = jnp.stack(os, axis=0)

    o1 = o[:, :, :_STARTS[1]].reshape(_B, _C, *_LVL_DIMS[0])
    o2 = o[:, :, _STARTS[1]:_STARTS[2]].reshape(_B, _C, *_LVL_DIMS[1])
    o3 = o[:, :, _STARTS[2]:].reshape(_B, _C, *_LVL_DIMS[2])
    return (o1, o2, o3)


# speedup vs baseline: 1.9010x; 1.9010x over previous
import functools

import jax
import jax.numpy as jnp
from jax import lax
from jax.experimental import pallas as pl
from jax.experimental.pallas import tpu as pltpu
from jax.experimental.pallas import tpu_sc as plsc

_B = 4
_C = 128
_LVL_DIMS = ((64, 64), (32, 32), (16, 16))
_LVL_SIZES = tuple(h * w for h, w in _LVL_DIMS)
_LEN = sum(_LVL_SIZES)
_STARTS = (0, _LVL_SIZES[0], _LVL_SIZES[0] + _LVL_SIZES[1])
_NH = 16
_NLVL = 3
_NPT = 4
_DH = _C // _NH
_NK = _NLVL * _NPT

_NQ = 128
_NCORES = 2
_NSUB = 16
_NW = _NCORES * _NSUB

_RPH = 48
_NROWS = _NH * _RPH


def _proj_kernel(x_ref, w_ref, b_ref, o_ref):
    y = jnp.dot(w_ref[...], x_ref[...], preferred_element_type=jnp.float32)
    o_ref[...] = y + b_ref[:, 0:1]


def _tc_project(x, w, bias2d, n_out_rows):
    nq_tile = 384
    return pl.pallas_call(
        _proj_kernel,
        grid=(_LEN // nq_tile,),
        in_specs=[
            pl.BlockSpec((_C, nq_tile), lambda j: (0, j)),
            pl.BlockSpec((n_out_rows, _C), lambda j: (0, 0)),
            pl.BlockSpec((n_out_rows, 128), lambda j: (0, 0)),
        ],
        out_specs=pl.BlockSpec((n_out_rows, nq_tile), lambda j: (0, j)),
        out_shape=jax.ShapeDtypeStruct((n_out_rows, _LEN), jnp.float32),
    )(x, w, bias2d)


def _out_kernel(x_ref, w_ref, b_ref, o_ref):
    y = jnp.dot(w_ref[...], x_ref[...], preferred_element_type=jnp.float32)
    o_ref[...] = y + b_ref[:, 0:1]


def _tc_out(x, w, bias2d):
    nq_tile = 384
    return pl.pallas_call(
        _out_kernel,
        grid=(_LEN // nq_tile,),
        in_specs=[
            pl.BlockSpec((_C, nq_tile), lambda j: (0, j)),
            pl.BlockSpec((_C, _C), lambda j: (0, 0)),
            pl.BlockSpec((_C, 128), lambda j: (0, 0)),
        ],
        out_specs=pl.BlockSpec((_C, nq_tile), lambda j: (0, j)),
        out_shape=jax.ShapeDtypeStruct((_C, _LEN), jnp.float32),
    )(x, w, bias2d)


def _sc_body(y_hbm, out_hbm, vtab, oa_v, out_v, in_sem, out_sem):
    wid = lax.axis_index("s") * _NCORES + lax.axis_index("c")
    h = wid // 2
    half = wid % 2

    def in_copy(h, q0, slot):
        return pltpu.make_async_copy(
            y_hbm.at[pl.ds(_RPH * h + _DH, 40), pl.ds(q0, _NQ)],
            oa_v.at[slot], in_sem)

    def out_copy(h, q0, slot):
        return pltpu.make_async_copy(
            out_v.at[slot], out_hbm.at[pl.ds(_DH * h, _DH), pl.ds(q0, _NQ)],
            out_sem)

    if True:
        pltpu.sync_copy(y_hbm.at[pl.ds(_RPH * h, _DH), :], vtab)

        for lq in range(_NLVL):
            hq, wq = _LVL_DIMS[lq]
            half_sz = _LVL_SIZES[lq] // 2
            n_chunks = half_sz // _NQ
            lg2w = wq.bit_length() - 1
            rel0 = half * half_sz

            in_copy(h, _STARTS[lq] + rel0, 0).start()

            def chunk_body(ci, carry2, lq=lq, hq=hq, wq=wq, lg2w=lg2w,
                           h=h, rel0=rel0, n_chunks=n_chunks):
                q0 = _STARTS[lq] + rel0 + ci * _NQ
                slot = lax.rem(ci, 2)

                @pl.when(ci + 1 < n_chunks)
                def _():
                    in_copy(h, q0 + _NQ, 1 - slot).start()

                in_copy(h, q0, slot).wait()

                @pl.when(ci >= 2)
                def _():
                    out_copy(h, q0 - 2 * _NQ, slot).wait()

                def do_group(s0):
                    rel = rel0 + ci * _NQ + s0 + lax.iota(jnp.int32, 16)
                    xq = rel & (wq - 1)
                    yq = lax.shift_right_logical(rel, lg2w)
                    xf = xq.astype(jnp.float32)
                    yf = yq.astype(jnp.float32)

                    avs = [oa_v[slot, 24 + k, pl.ds(s0, 16)] for k in range(_NK)]
                    m = avs[0]
                    for k in range(1, _NK):
                        m = jnp.maximum(m, avs[k])
                    es = [jnp.exp(a - m) for a in avs]
                    ssum = es[0]
                    for k in range(1, _NK):
                        ssum = ssum + es[k]
                    rs = 1.0 / ssum

                    acc = [jnp.zeros((16,), jnp.float32) for _ in range(_DH)]
                    for k in range(_NK):
                        l = k // _NPT
                        hl, wl = _LVL_DIMS[l]
                        base = _STARTS[l]
                        sx = wl / wq
                        sy = hl / hq
                        gx = xf * sx + (0.5 * sx - 0.5) + oa_v[slot, k, pl.ds(s0, 16)]
                        gy = yf * sy + (0.5 * sy - 0.5) + oa_v[slot, 12 + k, pl.ds(s0, 16)]

                        x0t = gx.astype(jnp.int32)
                        x0i = x0t - (x0t.astype(jnp.float32) > gx).astype(jnp.int32)
                        y0t = gy.astype(jnp.int32)
                        y0i = y0t - (y0t.astype(jnp.float32) > gy).astype(jnp.int32)
                        fx = gx - x0i.astype(jnp.float32)
                        fy = gy - y0i.astype(jnp.float32)

                        a = es[k] * rs
                        zero = jnp.zeros((16,), jnp.float32)
                        wx0 = jnp.where((x0i >= 0) & (x0i <= wl - 1), 1.0 - fx, zero)
                        wx1 = jnp.where((x0i >= -1) & (x0i < wl - 1), fx, zero)
                        wy0 = jnp.where((y0i >= 0) & (y0i <= hl - 1), (1.0 - fy) * a, zero)
                        wy1 = jnp.where((y0i >= -1) & (y0i < hl - 1), fy * a, zero)

                        xc0 = jnp.clip(x0i, 0, wl - 1)
                        xc1 = jnp.clip(x0i + 1, 0, wl - 1)
                        yc0 = jnp.clip(y0i, 0, hl - 1)
                        yc1 = jnp.clip(y0i + 1, 0, hl - 1)
                        row0 = base + yc0 * wl
                        row1 = base + yc1 * wl
                        i00 = row0 + xc0
                        i01 = row0 + xc1
                        i10 = row1 + xc0
                        i11 = row1 + xc1
                        w00 = wx0 * wy0
                        w01 = wx1 * wy0
                        w10 = wx0 * wy1
                        w11 = wx1 * wy1

                        for ch in range(_DH):
                            chv = jnp.full((16,), ch, jnp.int32)
                            g00 = plsc.load_gather(vtab, [chv, i00])
                            g01 = plsc.load_gather(vtab, [chv, i01])
                            g10 = plsc.load_gather(vtab, [chv, i10])
                            g11 = plsc.load_gather(vtab, [chv, i11])
                            acc[ch] = acc[ch] + (g00 * w00 + g01 * w01 +
                                                 g10 * w10 + g11 * w11)

                    for ch in range(_DH):
                        out_v[slot, ch, pl.ds(s0, 16)] = acc[ch]

                def sub_body(si, carry3):
                    do_group(si * 16)
                    return carry3

                lax.fori_loop(0, _NQ // 16, sub_body, 0)
                out_copy(h, q0, slot).start()
                return carry2

            lax.fori_loop(0, n_chunks, chunk_body, 0)
            for d in range(min(2, n_chunks)):
                out_copy(h, _STARTS[lq], 0).wait()


def _sc_sample(y):
    mesh = plsc.VectorSubcoreMesh(
        core_axis_name="c", subcore_axis_name="s",
        num_cores=_NCORES, num_subcores=_NSUB)
    k = pl.kernel(
        _sc_body,
        out_type=jax.ShapeDtypeStruct((_C, _LEN), jnp.float32),
        mesh=mesh,
        compiler_params=pltpu.CompilerParams(needs_layout_passes=False),
        scratch_types=[
            pltpu.VMEM((_DH, _LEN), jnp.float32),
            pltpu.VMEM((2, 40, _NQ), jnp.float32),
            pltpu.VMEM((2, _DH, _NQ), jnp.float32),
            pltpu.SemaphoreType.DMA,
            pltpu.SemaphoreType.DMA,
        ],
    )
    return k(y)


def kernel(feature_map1, feature_map2, feature_map3, W_value, b_value,
           W_off, b_off, W_attn, b_attn, W_out, b_out):
    x = jnp.concatenate(
        [feature_map1.reshape(_B, _C, -1),
         feature_map2.reshape(_B, _C, -1),
         feature_map3.reshape(_B, _C, -1)], axis=2)

    w_off_r = W_off.reshape(_NH, _NK, 2, _C)
    b_off_r = b_off.reshape(_NH, _NK, 2)
    w_all = jnp.concatenate(
        [W_value.reshape(_NH, _DH, _C),
         w_off_r[:, :, 0, :], w_off_r[:, :, 1, :],
         W_attn.reshape(_NH, _NK, _C),
         jnp.zeros((_NH, 4, _C), jnp.float32)], axis=1).reshape(_NROWS, _C)
    b_all = jnp.concatenate(
        [b_value.reshape(_NH, _DH),
         b_off_r[:, :, 0], b_off_r[:, :, 1],
         b_attn.reshape(_NH, _NK),
         jnp.zeros((_NH, 4), jnp.float32)], axis=1).reshape(_NROWS)
    bias2d = jnp.tile(b_all[:, None], (1, 128))

    bias_out = jnp.tile(b_out[:, None], (1, 128))
    os = []
    for b in range(_B):
        y_b = _tc_project(x[b], w_all, bias2d, _NROWS)
        s_b = _sc_sample(y_b)
        os.append(_tc_out(s_b, W_out, bias_out))
    o = jnp.stack(os, axis=0)

    o1 = o[:, :, :_STARTS[1]].reshape(_B, _C, *_LVL_DIMS[0])
    o2 = o[:, :, _STARTS[1]:_STARTS[2]].reshape(_B, _C, *_LVL_DIMS[1])
    o3 = o[:, :, _STARTS[2]:].reshape(_B, _C, *_LVL_DIMS[2])
    return (o1, o2, o3)


# speedup vs baseline: 1.9574x; 1.0297x over previous
import functools

import jax
import jax.numpy as jnp
from jax import lax
from jax.experimental import pallas as pl
from jax.experimental.pallas import tpu as pltpu
from jax.experimental.pallas import tpu_sc as plsc

_B = 4
_C = 128
_LVL_DIMS = ((64, 64), (32, 32), (16, 16))
_LVL_SIZES = tuple(h * w for h, w in _LVL_DIMS)
_LEN = sum(_LVL_SIZES)
_STARTS = (0, _LVL_SIZES[0], _LVL_SIZES[0] + _LVL_SIZES[1])
_NH = 16
_NLVL = 3
_NPT = 4
_DH = _C // _NH
_NK = _NLVL * _NPT

_NQ = 128
_NCORES = 2
_NSUB = 16
_NW = _NCORES * _NSUB

_RPH = 48
_NROWS = _NH * _RPH


def _proj_kernel(x_ref, w_ref, b_ref, o_ref):
    y = jnp.dot(w_ref[0], x_ref[0], preferred_element_type=jnp.float32)
    o_ref[0] = y + b_ref[:, 0:1]


def _tc_project(x, w, bias2d, n_out_rows):
    nq_tile = 384
    return pl.pallas_call(
        _proj_kernel,
        grid=(2, _LEN // nq_tile),
        in_specs=[
            pl.BlockSpec((1, _C, nq_tile), lambda b, j: (b, 0, j)),
            pl.BlockSpec((1, n_out_rows, _C), lambda b, j: (0, 0, 0)),
            pl.BlockSpec((n_out_rows, 128), lambda b, j: (0, 0)),
        ],
        out_specs=pl.BlockSpec((1, n_out_rows, nq_tile), lambda b, j: (b, 0, j)),
        out_shape=jax.ShapeDtypeStruct((2, n_out_rows, _LEN), jnp.float32),
    )(x, w[None], bias2d)


def _sc_body(y_hbm, out_hbm, vtab, oa_v, out_v, in_sem, out_sem):
    wid = lax.axis_index("s") * _NCORES + lax.axis_index("c")
    h = wid % _NH
    bl = wid // _NH

    def in_copy(bl, h, q0, slot):
        return pltpu.make_async_copy(
            y_hbm.at[bl, pl.ds(_RPH * h + _DH, 40), pl.ds(q0, _NQ)],
            oa_v.at[slot], in_sem)

    def out_copy(bl, h, q0, slot):
        return pltpu.make_async_copy(
            out_v.at[slot], out_hbm.at[bl, pl.ds(_DH * h, _DH), pl.ds(q0, _NQ)],
            out_sem)

    if True:
        pltpu.sync_copy(y_hbm.at[bl, pl.ds(_RPH * h, _DH), :], vtab)

        for lq in range(_NLVL):
            hq, wq = _LVL_DIMS[lq]
            n_chunks = _LVL_SIZES[lq] // _NQ
            lg2w = wq.bit_length() - 1
            rel0 = 0

            in_copy(bl, h, _STARTS[lq], 0).start()

            def chunk_body(ci, carry2, lq=lq, hq=hq, wq=wq, lg2w=lg2w,
                           bl=bl, h=h, rel0=rel0, n_chunks=n_chunks):
                q0 = _STARTS[lq] + ci * _NQ
                slot = lax.rem(ci, 2)

                @pl.when(ci + 1 < n_chunks)
                def _():
                    in_copy(bl, h, q0 + _NQ, 1 - slot).start()

                in_copy(bl, h, q0, slot).wait()

                @pl.when(ci >= 2)
                def _():
                    out_copy(bl, h, q0 - 2 * _NQ, slot).wait()

                def do_group(s0):
                    rel = rel0 + ci * _NQ + s0 + lax.iota(jnp.int32, 16)
                    xq = rel & (wq - 1)
                    yq = lax.shift_right_logical(rel, lg2w)
                    xf = xq.astype(jnp.float32)
                    yf = yq.astype(jnp.float32)

                    avs = [oa_v[slot, 24 + k, pl.ds(s0, 16)] for k in range(_NK)]
                    m = avs[0]
                    for k in range(1, _NK):
                        m = jnp.maximum(m, avs[k])
                    es = [jnp.exp(a - m) for a in avs]
                    ssum = es[0]
                    for k in range(1, _NK):
                        ssum = ssum + es[k]
                    rs = 1.0 / ssum

                    acc = [jnp.zeros((16,), jnp.float32) for _ in range(_DH)]
                    for k in range(_NK):
                        l = k // _NPT
                        hl, wl = _LVL_DIMS[l]
                        base = _STARTS[l]
                        sx = wl / wq
                        sy = hl / hq
                        gx = xf * sx + (0.5 * sx - 0.5) + oa_v[slot, k, pl.ds(s0, 16)]
                        gy = yf * sy + (0.5 * sy - 0.5) + oa_v[slot, 12 + k, pl.ds(s0, 16)]

                        x0t = gx.astype(jnp.int32)
                        x0i = x0t - (x0t.astype(jnp.float32) > gx).astype(jnp.int32)
                        y0t = gy.astype(jnp.int32)
                        y0i = y0t - (y0t.astype(jnp.float32) > gy).astype(jnp.int32)
                        fx = gx - x0i.astype(jnp.float32)
                        fy = gy - y0i.astype(jnp.float32)

                        a = es[k] * rs
                        zero = jnp.zeros((16,), jnp.float32)
                        wx0 = jnp.where((x0i >= 0) & (x0i <= wl - 1), 1.0 - fx, zero)
                        wx1 = jnp.where((x0i >= -1) & (x0i < wl - 1), fx, zero)
                        wy0 = jnp.where((y0i >= 0) & (y0i <= hl - 1), (1.0 - fy) * a, zero)
                        wy1 = jnp.where((y0i >= -1) & (y0i < hl - 1), fy * a, zero)

                        xc0 = jnp.clip(x0i, 0, wl - 1)
                        xc1 = jnp.clip(x0i + 1, 0, wl - 1)
                        yc0 = jnp.clip(y0i, 0, hl - 1)
                        yc1 = jnp.clip(y0i + 1, 0, hl - 1)
                        row0 = base + yc0 * wl
                        row1 = base + yc1 * wl
                        i00 = row0 + xc0
                        i01 = row0 + xc1
                        i10 = row1 + xc0
                        i11 = row1 + xc1
                        w00 = wx0 * wy0
                        w01 = wx1 * wy0
                        w10 = wx0 * wy1
                        w11 = wx1 * wy1

                        for ch in range(_DH):
                            chv = jnp.full((16,), ch, jnp.int32)
                            g00 = plsc.load_gather(vtab, [chv, i00])
                            g01 = plsc.load_gather(vtab, [chv, i01])
                            g10 = plsc.load_gather(vtab, [chv, i10])
                            g11 = plsc.load_gather(vtab, [chv, i11])
                            acc[ch] = acc[ch] + (g00 * w00 + g01 * w01 +
                                                 g10 * w10 + g11 * w11)

                    for ch in range(_DH):
                        out_v[slot, ch, pl.ds(s0, 16)] = acc[ch]

                def sub_body(si, carry3):
                    do_group(si * 16)
                    return carry3

                lax.fori_loop(0, _NQ // 16, sub_body, 0)
                out_copy(bl, h, q0, slot).start()
                return carry2

            lax.fori_loop(0, n_chunks, chunk_body, 0)
            for d in range(min(2, n_chunks)):
                out_copy(bl, h, _STARTS[lq], 0).wait()


def _sc_sample(y):
    mesh = plsc.VectorSubcoreMesh(
        core_axis_name="c", subcore_axis_name="s",
        num_cores=_NCORES, num_subcores=_NSUB)
    k = pl.kernel(
        _sc_body,
        out_type=jax.ShapeDtypeStruct((2, _C, _LEN), jnp.float32),
        mesh=mesh,
        compiler_params=pltpu.CompilerParams(needs_layout_passes=False),
        scratch_types=[
            pltpu.VMEM((_DH, _LEN), jnp.float32),
            pltpu.VMEM((2, 40, _NQ), jnp.float32),
            pltpu.VMEM((2, _DH, _NQ), jnp.float32),
            pltpu.SemaphoreType.DMA,
            pltpu.SemaphoreType.DMA,
        ],
    )
    return k(y)


def kernel(feature_map1, feature_map2, feature_map3, W_value, b_value,
           W_off, b_off, W_attn, b_attn, W_out, b_out):
    x = jnp.concatenate(
        [feature_map1.reshape(_B, _C, -1),
         feature_map2.reshape(_B, _C, -1),
         feature_map3.reshape(_B, _C, -1)], axis=2)

    w_off_r = W_off.reshape(_NH, _NK, 2, _C)
    b_off_r = b_off.reshape(_NH, _NK, 2)
    w_all = jnp.concatenate(
        [W_value.reshape(_NH, _DH, _C),
         w_off_r[:, :, 0, :], w_off_r[:, :, 1, :],
         W_attn.reshape(_NH, _NK, _C),
         jnp.zeros((_NH, 4, _C), jnp.float32)], axis=1).reshape(_NROWS, _C)
    b_all = jnp.concatenate(
        [b_value.reshape(_NH, _DH),
         b_off_r[:, :, 0], b_off_r[:, :, 1],
         b_attn.reshape(_NH, _NK),
         jnp.zeros((_NH, 4), jnp.float32)], axis=1).reshape(_NROWS)
    bias2d = jnp.tile(b_all[:, None], (1, 128))

    bias_out = jnp.tile(b_out[:, None], (1, 128))
    os = []
    for p in range(_B // 2):
        y_p = _tc_project(x[2 * p:2 * p + 2], w_all, bias2d, _NROWS)
        s_p = _sc_sample(y_p)
        os.append(_tc_project(s_p, W_out, bias_out, _C))
    o = jnp.concatenate(os, axis=0)

    o1 = o[:, :, :_STARTS[1]].reshape(_B, _C, *_LVL_DIMS[0])
    o2 = o[:, :, _STARTS[1]:_STARTS[2]].reshape(_B, _C, *_LVL_DIMS[1])
    o3 = o[:, :, _STARTS[2]:].reshape(_B, _C, *_LVL_DIMS[2])
    return (o1, o2, o3)


# speedup vs baseline: 1.9589x; 1.0008x over previous
import functools

import jax
import jax.numpy as jnp
from jax import lax
from jax.experimental import pallas as pl
from jax.experimental.pallas import tpu as pltpu
from jax.experimental.pallas import tpu_sc as plsc

_B = 4
_C = 128
_LVL_DIMS = ((64, 64), (32, 32), (16, 16))
_LVL_SIZES = tuple(h * w for h, w in _LVL_DIMS)
_LEN = sum(_LVL_SIZES)
_STARTS = (0, _LVL_SIZES[0], _LVL_SIZES[0] + _LVL_SIZES[1])
_NH = 16
_NLVL = 3
_NPT = 4
_DH = _C // _NH
_NK = _NLVL * _NPT

_NQ = 128
_NCORES = 2
_NSUB = 16
_NW = _NCORES * _NSUB

_RPH = 48
_NROWS = _NH * _RPH


def _proj_kernel(x_ref, w_ref, b_ref, o_ref):
    y = jnp.dot(w_ref[0], x_ref[0], preferred_element_type=jnp.float32)
    o_ref[0] = y + b_ref[:, 0:1]


def _tc_project(x, w, bias2d, n_out_rows):
    nq_tile = 384
    return pl.pallas_call(
        _proj_kernel,
        grid=(2, _LEN // nq_tile),
        in_specs=[
            pl.BlockSpec((1, _C, nq_tile), lambda b, j: (b, 0, j)),
            pl.BlockSpec((1, n_out_rows, _C), lambda b, j: (0, 0, 0)),
            pl.BlockSpec((n_out_rows, 128), lambda b, j: (0, 0)),
        ],
        out_specs=pl.BlockSpec((1, n_out_rows, nq_tile), lambda b, j: (b, 0, j)),
        out_shape=jax.ShapeDtypeStruct((2, n_out_rows, _LEN), jnp.float32),
    )(x, w[None], bias2d)


def _sc_body(y_hbm, out_hbm, vtab, oa_v, out_v, in_sem, out_sem):
    wid = lax.axis_index("s") * _NCORES + lax.axis_index("c")
    h = wid % _NH
    bl = wid // _NH

    def in_copy(bl, h, q0, slot):
        return pltpu.make_async_copy(
            y_hbm.at[bl, pl.ds(_RPH * h + _DH, 40), pl.ds(q0, _NQ)],
            oa_v.at[slot], in_sem)

    def out_copy(bl, h, q0, slot):
        return pltpu.make_async_copy(
            out_v.at[slot], out_hbm.at[bl, pl.ds(_DH * h, _DH), pl.ds(q0, _NQ)],
            out_sem)

    pltpu.sync_copy(y_hbm.at[bl, pl.ds(_RPH * h, _DH), :], vtab)

    for lq in range(_NLVL):
        hq, wq = _LVL_DIMS[lq]
        n_chunks = _LVL_SIZES[lq] // _NQ
        lg2w = wq.bit_length() - 1
        rel0 = 0

        in_copy(bl, h, _STARTS[lq], 0).start()

        def chunk_body(ci, carry2, lq=lq, hq=hq, wq=wq, lg2w=lg2w,
                       bl=bl, h=h, rel0=rel0, n_chunks=n_chunks):
            q0 = _STARTS[lq] + ci * _NQ
            slot = lax.rem(ci, 2)

            @pl.when(ci + 1 < n_chunks)
            def _():
                in_copy(bl, h, q0 + _NQ, 1 - slot).start()

            in_copy(bl, h, q0, slot).wait()

            @pl.when(ci >= 2)
            def _():
                out_copy(bl, h, q0 - 2 * _NQ, slot).wait()

            def do_group(s0):
                rel = rel0 + ci * _NQ + s0 + lax.iota(jnp.int32, 16)
                xq = rel & (wq - 1)
                yq = lax.shift_right_logical(rel, lg2w)
                xf = xq.astype(jnp.float32)
                yf = yq.astype(jnp.float32)

                avs = [oa_v[slot, 24 + k, pl.ds(s0, 16)] for k in range(_NK)]
                m = avs[0]
                for k in range(1, _NK):
                    m = jnp.maximum(m, avs[k])
                es = [jnp.exp(a - m) for a in avs]
                ssum = es[0]
                for k in range(1, _NK):
                    ssum = ssum + es[k]
                rs = 1.0 / ssum

                acc = [jnp.zeros((16,), jnp.float32) for _ in range(_DH)]
                for k in range(_NK):
                    l = k // _NPT
                    hl, wl = _LVL_DIMS[l]
                    base = _STARTS[l]
                    sx = wl / wq
                    sy = hl / hq
                    gx = xf * sx + (0.5 * sx - 0.5) + oa_v[slot, k, pl.ds(s0, 16)]
                    gy = yf * sy + (0.5 * sy - 0.5) + oa_v[slot, 12 + k, pl.ds(s0, 16)]

                    x0t = gx.astype(jnp.int32)
                    x0i = x0t - (x0t.astype(jnp.float32) > gx).astype(jnp.int32)
                    y0t = gy.astype(jnp.int32)
                    y0i = y0t - (y0t.astype(jnp.float32) > gy).astype(jnp.int32)
                    fx = gx - x0i.astype(jnp.float32)
                    fy = gy - y0i.astype(jnp.float32)

                    a = es[k] * rs
                    zero = jnp.zeros((16,), jnp.float32)
                    wx0 = jnp.where((x0i >= 0) & (x0i <= wl - 1), 1.0 - fx, zero)
                    wx1 = jnp.where((x0i >= -1) & (x0i < wl - 1), fx, zero)
                    wy0 = jnp.where((y0i >= 0) & (y0i <= hl - 1), (1.0 - fy) * a, zero)
                    wy1 = jnp.where((y0i >= -1) & (y0i < hl - 1), fy * a, zero)

                    xc0 = jnp.clip(x0i, 0, wl - 1)
                    xc1 = jnp.clip(x0i + 1, 0, wl - 1)
                    yc0 = jnp.clip(y0i, 0, hl - 1)
                    yc1 = jnp.clip(y0i + 1, 0, hl - 1)
                    row0 = base + yc0 * wl
                    row1 = base + yc1 * wl
                    i00 = row0 + xc0
                    i01 = row0 + xc1
                    i10 = row1 + xc0
                    i11 = row1 + xc1
                    w00 = wx0 * wy0
                    w01 = wx1 * wy0
                    w10 = wx0 * wy1
                    w11 = wx1 * wy1

                    for ch in range(_DH):
                        chv = jnp.full((16,), ch, jnp.int32)
                        g00 = plsc.load_gather(vtab, [chv, i00])
                        g01 = plsc.load_gather(vtab, [chv, i01])
                        g10 = plsc.load_gather(vtab, [chv, i10])
                        g11 = plsc.load_gather(vtab, [chv, i11])
                        acc[ch] = acc[ch] + (g00 * w00 + g01 * w01 +
                                             g10 * w10 + g11 * w11)

                for ch in range(_DH):
                    out_v[slot, ch, pl.ds(s0, 16)] = acc[ch]

            def sub_body(si, carry3):
                do_group(si * 16)
                return carry3

            lax.fori_loop(0, _NQ // 16, sub_body, 0)
            out_copy(bl, h, q0, slot).start()
            return carry2

        lax.fori_loop(0, n_chunks, chunk_body, 0)
        for d in range(min(2, n_chunks)):
            out_copy(bl, h, _STARTS[lq], 0).wait()


def _sc_sample(y):
    mesh = plsc.VectorSubcoreMesh(
        core_axis_name="c", subcore_axis_name="s",
        num_cores=_NCORES, num_subcores=_NSUB)
    k = pl.kernel(
        _sc_body,
        out_type=jax.ShapeDtypeStruct((2, _C, _LEN), jnp.float32),
        mesh=mesh,
        compiler_params=pltpu.CompilerParams(needs_layout_passes=False),
        scratch_types=[
            pltpu.VMEM((_DH, _LEN), jnp.float32),
            pltpu.VMEM((2, 40, _NQ), jnp.float32),
            pltpu.VMEM((2, _DH, _NQ), jnp.float32),
            pltpu.SemaphoreType.DMA,
            pltpu.SemaphoreType.DMA,
        ],
    )
    return k(y)


def kernel(feature_map1, feature_map2, feature_map3, W_value, b_value,
           W_off, b_off, W_attn, b_attn, W_out, b_out):
    x = jnp.concatenate(
        [feature_map1.reshape(_B, _C, -1),
         feature_map2.reshape(_B, _C, -1),
         feature_map3.reshape(_B, _C, -1)], axis=2)

    w_off_r = W_off.reshape(_NH, _NK, 2, _C)
    b_off_r = b_off.reshape(_NH, _NK, 2)
    w_all = jnp.concatenate(
        [W_value.reshape(_NH, _DH, _C),
         w_off_r[:, :, 0, :], w_off_r[:, :, 1, :],
         W_attn.reshape(_NH, _NK, _C),
         jnp.zeros((_NH, 4, _C), jnp.float32)], axis=1).reshape(_NROWS, _C)
    b_all = jnp.concatenate(
        [b_value.reshape(_NH, _DH),
         b_off_r[:, :, 0], b_off_r[:, :, 1],
         b_attn.reshape(_NH, _NK),
         jnp.zeros((_NH, 4), jnp.float32)], axis=1).reshape(_NROWS)
    bias2d = jnp.tile(b_all[:, None], (1, 128))

    bias_out = jnp.tile(b_out[:, None], (1, 128))
    os = []
    for p in range(_B // 2):
        y_p = _tc_project(x[2 * p:2 * p + 2], w_all, bias2d, _NROWS)
        s_p = _sc_sample(y_p)
        os.append(_tc_project(s_p, W_out, bias_out, _C))
    o = jnp.concatenate(os, axis=0)

    o1 = o[:, :, :_STARTS[1]].reshape(_B, _C, *_LVL_DIMS[0])
    o2 = o[:, :, _STARTS[1]:_STARTS[2]].reshape(_B, _C, *_LVL_DIMS[1])
    o3 = o[:, :, _STARTS[2]:].reshape(_B, _C, *_LVL_DIMS[2])
    return (o1, o2, o3)
